# baseline (speedup 1.0000x reference)
import functools

import numpy as np
import jax
import jax.numpy as jnp
from jax.experimental import pallas as pl
from jax.experimental.pallas import tpu as pltpu

_NEG = np.float32(-1e9)


def _hn_body(x_ref, w0_ref, b0_ref, w1_ref, b1_ref, hn0_ref, hn1_ref):
    x = x_ref[...]
    for w_ref, b_ref, o_ref in ((w0_ref, b0_ref, hn0_ref),
                                (w1_ref, b1_ref, hn1_ref)):
        h = jnp.dot(x, w_ref[...], preferred_element_type=jnp.float32) + b_ref[...]
        nrm = jnp.sqrt(jnp.sum(h * h, axis=-1, keepdims=True))
        o_ref[...] = h / (nrm + jnp.float32(1e-8))


def _layer_body(adj_ref, hn0_ref, hn1_ref, x_ref, out_ref, *, br, k):
    i = pl.program_id(0)
    adj = adj_ref[...]
    n = adj.shape[1]
    xb = x_ref[...].astype(jnp.bfloat16)
    ones8 = jnp.ones((n, 8), jnp.bfloat16)
    kf = jnp.float32(k)

    scores = []
    skeys = []
    for hn_ref in (hn0_ref, hn1_ref):
        hnf = hn_ref[...]
        hnb = hn_ref[pl.ds(i * br, br), :]
        sim = jax.lax.dot_general(hnb, hnf, (((1,), (1,)), ((), ())),
                                  preferred_element_type=jnp.float32)
        sc = jnp.where(adj > 0, sim, _NEG)
        mdom = jnp.maximum(sc + jnp.float32(26.0), jnp.float32(24.0))
        scores.append(sc)
        skeys.append(jax.lax.bitcast_convert_type(mdom, jnp.int32))

    accs = [jnp.full((br, 1), np.int32(0x41C00000), jnp.int32)
            for _ in range(2)]
    for bit in range(20, -1, -1):
        for h in range(2):
            cand = accs[h] | np.int32(1 << bit)
            cnt = jnp.sum((skeys[h] >= cand).astype(jnp.int32),
                          axis=-1, keepdims=True)
            accs[h] = jnp.where(cnt >= k, cand, accs[h])

    acc_out = None
    for h in range(2):
        sc = scores[h]
        mdom = jnp.maximum(sc + jnp.float32(26.0), jnp.float32(24.0))
        vt = jax.lax.bitcast_convert_type(accs[h], jnp.float32)
        mask = mdom >= vt
        m = jnp.max(sc, axis=-1, keepdims=True)
        p = jnp.where(mask, jnp.exp(sc - m), jnp.float32(0.0)).astype(jnp.bfloat16)
        s = jax.lax.dot_general(p, ones8, (((1,), (0,)), ((), ())),
                                preferred_element_type=jnp.float32)
        o = jax.lax.dot_general(p, xb, (((1,), (0,)), ((), ())),
                                preferred_element_type=jnp.float32)
        o = o / s[:, 0:1]
        acc_out = o if acc_out is None else acc_out + o
    out_ref[...] = acc_out * jnp.float32(0.5)


def _layer(x, adj, W0, b0, W1, b1, br):
    n, d = x.shape
    hid = W0.shape[1]
    hn0, hn1 = pl.pallas_call(
        _hn_body,
        out_shape=[jax.ShapeDtypeStruct((n, hid), jnp.float32)] * 2,
    )(x, W0, b0.reshape(1, hid), W1, b1.reshape(1, hid))
    k = max(1, int(0.5 * n))
    out = pl.pallas_call(
        functools.partial(_layer_body, br=br, k=k),
        grid=(n // br,),
        in_specs=[
            pl.BlockSpec((br, n), lambda i: (i, 0)),
            pl.BlockSpec((n, hid), lambda i: (0, 0)),
            pl.BlockSpec((n, hid), lambda i: (0, 0)),
            pl.BlockSpec((n, d), lambda i: (0, 0)),
        ],
        out_specs=pl.BlockSpec((br, d), lambda i: (i, 0)),
        out_shape=jax.ShapeDtypeStruct((n, d), jnp.float32),
        compiler_params=pltpu.CompilerParams(
            dimension_semantics=("parallel",)),
    )(adj, hn0, hn1, x)
    return out


def kernel(features, adj0, adj1, W_0_0, b_0_0, W_0_1, b_0_1,
           W_1_0, b_1_0, W_1_1, b_1_1):
    x = _layer(features, adj0, W_0_0, b_0_0, W_0_1, b_0_1, 512)
    x = _layer(x, adj1, W_1_0, b_1_0, W_1_1, b_1_1, 512)
    return x


# speedup vs baseline: 1.1628x; 1.1628x over previous
import functools

import numpy as np
import jax
import jax.numpy as jnp
from jax.experimental import pallas as pl
from jax.experimental.pallas import tpu as pltpu

_NEG = np.float32(-1e9)


def _hn_body(x_ref, w0_ref, b0_ref, w1_ref, b1_ref, hn0_ref, hn1_ref):
    x = x_ref[...]
    for w_ref, b_ref, o_ref in ((w0_ref, b0_ref, hn0_ref),
                                (w1_ref, b1_ref, hn1_ref)):
        h = jnp.dot(x, w_ref[...], preferred_element_type=jnp.float32) + b_ref[...]
        nrm = jnp.sqrt(jnp.sum(h * h, axis=-1, keepdims=True))
        o_ref[...] = h / (nrm + jnp.float32(1e-8))


def _layer_body(adj_ref, hn0_ref, hn1_ref, x_ref, out_ref, *, br, k):
    i = pl.program_id(0)
    adj = adj_ref[...]
    n = adj.shape[1]
    xb = x_ref[...].astype(jnp.bfloat16)
    ones8 = jnp.ones((n, 8), jnp.bfloat16)
    kf = jnp.float32(k)

    scores = []
    skeys = []
    for hn_ref in (hn0_ref, hn1_ref):
        hnf = hn_ref[...]
        hnb = hn_ref[pl.ds(i * br, br), :]
        sim = jax.lax.dot_general(hnb, hnf, (((1,), (1,)), ((), ())),
                                  preferred_element_type=jnp.float32)
        sc = jnp.where(adj > 0, sim, _NEG)
        mdom = jnp.maximum(sc + jnp.float32(26.0), jnp.float32(24.0))
        scores.append(sc)
        skeys.append(jax.lax.bitcast_convert_type(mdom, jnp.int32))

    accs = [jnp.full((br, 1), np.int32(0x41C00000), jnp.int32)
            for _ in range(2)]
    for bit in range(20, -1, -1):
        for h in range(2):
            cand = accs[h] | np.int32(1 << bit)
            cnt = jnp.sum((skeys[h] >= cand).astype(jnp.int32),
                          axis=-1, keepdims=True)
            accs[h] = jnp.where(cnt >= k, cand, accs[h])

    acc_out = None
    for h in range(2):
        sc = scores[h]
        mdom = jnp.maximum(sc + jnp.float32(26.0), jnp.float32(24.0))
        vt = jax.lax.bitcast_convert_type(accs[h], jnp.float32)
        mask = mdom >= vt
        m = jnp.max(sc, axis=-1, keepdims=True)
        p = jnp.where(mask, jnp.exp(sc - m), jnp.float32(0.0)).astype(jnp.bfloat16)
        s = jax.lax.dot_general(p, ones8, (((1,), (0,)), ((), ())),
                                preferred_element_type=jnp.float32)
        o = jax.lax.dot_general(p, xb, (((1,), (0,)), ((), ())),
                                preferred_element_type=jnp.float32)
        o = o / s[:, 0:1]
        acc_out = o if acc_out is None else acc_out + o
    out_ref[...] = acc_out * jnp.float32(0.5)


def _layer(x, adj, W0, b0, W1, b1, br):
    n, d = x.shape
    hid = W0.shape[1]
    hn0, hn1 = pl.pallas_call(
        _hn_body,
        out_shape=[jax.ShapeDtypeStruct((n, hid), jnp.float32)] * 2,
    )(x, W0, b0.reshape(1, hid), W1, b1.reshape(1, hid))
    k = max(1, int(0.5 * n))
    out = pl.pallas_call(
        functools.partial(_layer_body, br=br, k=k),
        grid=(n // br,),
        in_specs=[
            pl.BlockSpec((br, n), lambda i: (i, 0)),
            pl.BlockSpec((n, hid), lambda i: (0, 0)),
            pl.BlockSpec((n, hid), lambda i: (0, 0)),
            pl.BlockSpec((n, d), lambda i: (0, 0)),
        ],
        out_specs=pl.BlockSpec((br, d), lambda i: (i, 0)),
        out_shape=jax.ShapeDtypeStruct((n, d), jnp.float32),
        compiler_params=pltpu.CompilerParams(
            dimension_semantics=("parallel",)),
    )(adj, hn0, hn1, x)
    return out


def kernel(features, adj0, adj1, W_0_0, b_0_0, W_0_1, b_0_1,
           W_1_0, b_1_0, W_1_1, b_1_1):
    x = _layer(features, adj0, W_0_0, b_0_0, W_0_1, b_0_1, 128)
    x = _layer(x, adj1, W_1_0, b_1_0, W_1_1, b_1_1, 128)
    return x


# speedup vs baseline: 1.1744x; 1.0100x over previous
import functools

import numpy as np
import jax
import jax.numpy as jnp
from jax.experimental import pallas as pl
from jax.experimental.pallas import tpu as pltpu

_NEG = np.float32(-1e9)


def _layer_body(adj_ref, x_ref, w0_ref, b0_ref, w1_ref, b1_ref, out_ref,
                hn0_ref, hn1_ref, *, br, k):
    i = pl.program_id(0)

    @pl.when(i == 0)
    def _():
        x = x_ref[...]
        for w_ref, b_ref, o_ref in ((w0_ref, b0_ref, hn0_ref),
                                    (w1_ref, b1_ref, hn1_ref)):
            h = (jnp.dot(x, w_ref[...], preferred_element_type=jnp.float32)
                 + b_ref[...])
            nrm = jnp.sqrt(jnp.sum(h * h, axis=-1, keepdims=True))
            o_ref[...] = h / (nrm + jnp.float32(1e-8))

    adj = adj_ref[...]
    n = adj.shape[1]
    xb = x_ref[...].astype(jnp.bfloat16)
    ones8 = jnp.ones((n, 8), jnp.bfloat16)
    kf = jnp.float32(k)

    scores = []
    skeys = []
    for hn_ref in (hn0_ref, hn1_ref):
        hnf = hn_ref[...]
        hnb = hn_ref[pl.ds(i * br, br), :]
        sim = jax.lax.dot_general(hnb, hnf, (((1,), (1,)), ((), ())),
                                  preferred_element_type=jnp.float32)
        sc = jnp.where(adj > 0, sim, _NEG)
        mdom = jnp.maximum(sc + jnp.float32(26.0), jnp.float32(24.0))
        scores.append(sc)
        skeys.append(jax.lax.bitcast_convert_type(mdom, jnp.int32))

    accs = [jnp.full((br, 1), np.int32(0x41C00000), jnp.int32)
            for _ in range(2)]
    for bit in range(20, -1, -1):
        for h in range(2):
            cand = accs[h] | np.int32(1 << bit)
            cnt = jnp.sum((skeys[h] >= cand).astype(jnp.int32),
                          axis=-1, keepdims=True)
            accs[h] = jnp.where(cnt >= k, cand, accs[h])

    acc_out = None
    for h in range(2):
        sc = scores[h]
        mdom = jnp.maximum(sc + jnp.float32(26.0), jnp.float32(24.0))
        vt = jax.lax.bitcast_convert_type(accs[h], jnp.float32)
        mask = mdom >= vt
        m = jnp.max(sc, axis=-1, keepdims=True)
        p = jnp.where(mask, jnp.exp(sc - m), jnp.float32(0.0)).astype(jnp.bfloat16)
        s = jax.lax.dot_general(p, ones8, (((1,), (0,)), ((), ())),
                                preferred_element_type=jnp.float32)
        o = jax.lax.dot_general(p, xb, (((1,), (0,)), ((), ())),
                                preferred_element_type=jnp.float32)
        o = o / s[:, 0:1]
        acc_out = o if acc_out is None else acc_out + o
    out_ref[...] = acc_out * jnp.float32(0.5)


def _layer(x, adj, W0, b0, W1, b1, br):
    n, d = x.shape
    hid = W0.shape[1]
    k = max(1, int(0.5 * n))
    out = pl.pallas_call(
        functools.partial(_layer_body, br=br, k=k),
        grid=(n // br,),
        in_specs=[
            pl.BlockSpec((br, n), lambda i: (i, 0)),
            pl.BlockSpec((n, d), lambda i: (0, 0)),
            pl.BlockSpec((d, hid), lambda i: (0, 0)),
            pl.BlockSpec((1, hid), lambda i: (0, 0)),
            pl.BlockSpec((d, hid), lambda i: (0, 0)),
            pl.BlockSpec((1, hid), lambda i: (0, 0)),
        ],
        out_specs=pl.BlockSpec((br, d), lambda i: (i, 0)),
        out_shape=jax.ShapeDtypeStruct((n, d), jnp.float32),
        scratch_shapes=[pltpu.VMEM((n, hid), jnp.float32),
                        pltpu.VMEM((n, hid), jnp.float32)],
    )(adj, x, W0, b0.reshape(1, hid), W1, b1.reshape(1, hid))
    return out


def kernel(features, adj0, adj1, W_0_0, b_0_0, W_0_1, b_0_1,
           W_1_0, b_1_0, W_1_1, b_1_1):
    x = _layer(features, adj0, W_0_0, b_0_0, W_0_1, b_0_1, 256)
    x = _layer(x, adj1, W_1_0, b_1_0, W_1_1, b_1_1, 256)
    return x


# speedup vs baseline: 1.1754x; 1.0008x over previous
import functools

import numpy as np
import jax
import jax.numpy as jnp
from jax.experimental import pallas as pl
from jax.experimental.pallas import tpu as pltpu

_NEG = np.float32(-1e9)


def _layer_body(adj_ref, x_ref, w0_ref, b0_ref, w1_ref, b1_ref, out_ref,
                hn0_ref, hn1_ref, *, br, k):
    i = pl.program_id(0)

    @pl.when(i == 0)
    def _():
        x = x_ref[...]
        for w_ref, b_ref, o_ref in ((w0_ref, b0_ref, hn0_ref),
                                    (w1_ref, b1_ref, hn1_ref)):
            h = (jnp.dot(x, w_ref[...], preferred_element_type=jnp.float32)
                 + b_ref[...])
            nrm = jnp.sqrt(jnp.sum(h * h, axis=-1, keepdims=True))
            o_ref[...] = h / (nrm + jnp.float32(1e-8))

    adj = adj_ref[...]
    n = adj.shape[1]
    xb = x_ref[...].astype(jnp.bfloat16)
    ones8 = jnp.ones((n, 8), jnp.bfloat16)

    scores = []
    skeys = []
    for hn_ref in (hn0_ref, hn1_ref):
        hnf = hn_ref[...]
        hnb = hn_ref[pl.ds(i * br, br), :]
        sim = jax.lax.dot_general(hnb, hnf, (((1,), (1,)), ((), ())),
                                  preferred_element_type=jnp.float32)
        sc = jnp.where(adj > 0, sim, _NEG)
        mdom = jnp.maximum(sc + jnp.float32(26.0), jnp.float32(24.0))
        scores.append(sc)
        skeys.append(jax.lax.bitcast_convert_type(mdom, jnp.int32))

    accs = [jnp.full((br, 1), np.int32(0x41C00000), jnp.int32)
            for _ in range(2)]
    for bit in range(20, -1, -1):
        for h in range(2):
            cand = accs[h] | np.int32(1 << bit)
            cnt = jnp.sum((skeys[h] >= cand).astype(jnp.int32),
                          axis=-1, keepdims=True)
            accs[h] = jnp.where(cnt >= k, cand, accs[h])

    acc_out = None
    for h in range(2):
        sc = scores[h]
        mdom = jnp.maximum(sc + jnp.float32(26.0), jnp.float32(24.0))
        vt = jax.lax.bitcast_convert_type(accs[h], jnp.float32)
        mask = mdom >= vt
        m = jnp.max(sc, axis=-1, keepdims=True)
        p = jnp.where(mask, jnp.exp(sc - m), jnp.float32(0.0)).astype(jnp.bfloat16)
        s = jax.lax.dot_general(p, ones8, (((1,), (0,)), ((), ())),
                                preferred_element_type=jnp.float32)
        o = jax.lax.dot_general(p, xb, (((1,), (0,)), ((), ())),
                                preferred_element_type=jnp.float32)
        o = o / s[:, 0:1]
        acc_out = o if acc_out is None else acc_out + o
    out_ref[...] = acc_out * jnp.float32(0.5)


def _layer(x, adj, W0, b0, W1, b1, br):
    n, d = x.shape
    hid = W0.shape[1]
    k = max(1, int(0.5 * n))
    out = pl.pallas_call(
        functools.partial(_layer_body, br=br, k=k),
        grid=(n // br,),
        in_specs=[
            pl.BlockSpec((br, n), lambda i: (i, 0)),
            pl.BlockSpec((n, d), lambda i: (0, 0)),
            pl.BlockSpec((d, hid), lambda i: (0, 0)),
            pl.BlockSpec((1, hid), lambda i: (0, 0)),
            pl.BlockSpec((d, hid), lambda i: (0, 0)),
            pl.BlockSpec((1, hid), lambda i: (0, 0)),
        ],
        out_specs=pl.BlockSpec((br, d), lambda i: (i, 0)),
        out_shape=jax.ShapeDtypeStruct((n, d), jnp.float32),
        scratch_shapes=[pltpu.VMEM((n, hid), jnp.float32),
                        pltpu.VMEM((n, hid), jnp.float32)],
    )(adj, x, W0, b0.reshape(1, hid), W1, b1.reshape(1, hid))
    return out


def kernel(features, adj0, adj1, W_0_0, b_0_0, W_0_1, b_0_1,
           W_1_0, b_1_0, W_1_1, b_1_1):
    x = _layer(features, adj0, W_0_0, b_0_0, W_0_1, b_0_1, 256)
    x = _layer(x, adj1, W_1_0, b_1_0, W_1_1, b_1_1, 256)
    return x
